# baseline (speedup 1.0000x reference)
import functools

import jax
import jax.numpy as jnp
from jax import lax
from jax.experimental import pallas as pl
from jax.experimental.pallas import tpu as pltpu
from jax.experimental.pallas import tpu_sc as plsc

_B = 1024
_N = 100000
_L = 50


_RBLK = 1000
_NS = _N // _RBLK


def _tc_sum_body(x_ref, acc_ref):
    s = pl.program_id(0)

    @pl.when(s == 0)
    def _():
        acc_ref[...] = jnp.zeros_like(acc_ref)

    ex = jnp.exp(x_ref[...])
    parts = [ex[k * 8:(k + 1) * 8, :] for k in range(_RBLK // 8)]
    while len(parts) > 1:
        parts = [parts[i] + parts[i + 1] for i in range(0, len(parts) - 1, 2)] \
            + ([parts[-1]] if len(parts) % 2 else [])
    acc_ref[...] += parts[0]


_tc_sum = pl.pallas_call(
    _tc_sum_body,
    grid=(_NS,),
    in_specs=[pl.BlockSpec((_RBLK, _B), lambda s: (s, 0))],
    out_specs=pl.BlockSpec((8, _B), lambda s: (0, 0)),
    out_shape=jax.ShapeDtypeStruct((8, _B), jnp.float32),
    compiler_params=pltpu.CompilerParams(
        dimension_semantics=("arbitrary",)),
)


def _tc_tail_body(acc_ref, g_ref, nl_ref, lpt_ref):
    sum_exp = jnp.sum(acc_ref[...], axis=0, keepdims=True)
    g = g_ref[...]
    tails = g[0:_L, :]
    tgt = g[_L:_L + 1, :]
    e = jnp.exp(tails)
    r = lax.broadcasted_iota(jnp.int32, (_L, _L), 0)
    c = lax.broadcasted_iota(jnp.int32, (_L, _L), 1)
    tri = jnp.where(c <= r, 1.0, 0.0)
    cs = lax.dot_general(tri, e, (((1,), (0,)), ((), ())),
                         precision=lax.Precision.HIGHEST,
                         preferred_element_type=jnp.float32)
    etot = cs[_L - 1:_L, :]
    others = sum_exp - jnp.exp(tgt) - etot
    below_sum = jnp.sum(jnp.log(others + etot - cs + e), axis=0,
                        keepdims=True)
    above = jnp.sum(tails, axis=0, keepdims=True)
    lpt = above - below_sum
    nl_ref[...] = jnp.log(sum_exp) - tgt - lpt
    lpt_ref[...] = lpt


_tc_tail = pl.pallas_call(
    _tc_tail_body,
    out_shape=[
        jax.ShapeDtypeStruct((1, _B), jnp.float32),
        jax.ShapeDtypeStruct((1, _B), jnp.float32),
    ],
)


_NIDX = _B * (_L + 1)
_NC = 2
_NSUB = 16
_NW = _NC * _NSUB
_PERW = _NIDX // _NW
_NCH = _PERW // 16


def _sc_body(table, fidx_hbm, out_hbm, fidx_v, sel_v, sem):
    wid = lax.axis_index("s") * _NC + lax.axis_index("c")
    base = wid * _PERW
    pltpu.sync_copy(fidx_hbm.at[pl.ds(base, _PERW)], fidx_v)
    pltpu.async_copy(table.at[fidx_v], sel_v, sem).wait()
    pltpu.sync_copy(sel_v, out_hbm.at[pl.ds(base, _PERW)])


@functools.cache
def _sc_gather():
    return functools.partial(
        pl.kernel,
        mesh=plsc.VectorSubcoreMesh(core_axis_name="c", subcore_axis_name="s"),
        out_type=jax.ShapeDtypeStruct((_NIDX,), jnp.float32),
        scratch_types=[
            pltpu.VMEM((_PERW,), jnp.int32),
            pltpu.VMEM((_PERW,), jnp.float32),
            pltpu.SemaphoreType.DMA,
        ],
    )(_sc_body)


def kernel(output, target, tails):
    flat_view = (output.T.reshape(_N // 8, 8, _B // 128, 128)
                 .transpose(0, 2, 1, 3).reshape(-1))
    idx = jnp.concatenate([tails, target[:, None]], axis=1).astype(jnp.int32)
    i_b = jnp.arange(_B, dtype=jnp.int32)[:, None]
    phys = (((idx >> 3) * (_B // 128) + (i_b >> 7)) * 8 + (idx & 7)) * 128 \
        + (i_b & 127)
    g = _sc_gather()(flat_view, phys.T.reshape(-1))
    gt = g.reshape(_L + 1, _B)
    acc8 = _tc_sum(output.T)
    nl, lpt = _tc_tail(acc8, gt)
    return nl[0], lpt[0]


# speedup vs baseline: 1.1781x; 1.1781x over previous
import functools

import jax
import jax.numpy as jnp
from jax import lax
from jax.experimental import pallas as pl
from jax.experimental.pallas import tpu as pltpu
from jax.experimental.pallas import tpu_sc as plsc

_B = 1024
_N = 100000
_L = 50


_NSC = 16000
_R0 = _N - _NSC
_RBLK = 2000
_NS = _R0 // _RBLK


def _tc_sum_body(x_ref, acc_ref):
    s = pl.program_id(0)

    @pl.when(s == 0)
    def _():
        acc_ref[...] = jnp.zeros_like(acc_ref)

    ex = jnp.exp(x_ref[...])
    parts = [ex[k * 8:(k + 1) * 8, :] for k in range(_RBLK // 8)]
    while len(parts) > 1:
        parts = [parts[i] + parts[i + 1] for i in range(0, len(parts) - 1, 2)] \
            + ([parts[-1]] if len(parts) % 2 else [])
    acc_ref[...] += parts[0]


_tc_sum = pl.pallas_call(
    _tc_sum_body,
    grid=(_NS,),
    in_specs=[pl.BlockSpec((_RBLK, _B), lambda s: (s, 0))],
    out_specs=pl.BlockSpec((8, _B), lambda s: (0, 0)),
    out_shape=jax.ShapeDtypeStruct((8, _B), jnp.float32),
    compiler_params=pltpu.CompilerParams(
        dimension_semantics=("arbitrary",)),
)


def _tc_tail_body(acc_ref, part_ref, g_ref, nl_ref, lpt_ref):
    sum_exp = jnp.sum(acc_ref[...], axis=0, keepdims=True) \
        + jnp.sum(part_ref[...], axis=0, keepdims=True)
    g = g_ref[...]
    tails = g[0:_L, :]
    tgt = g[_L:_L + 1, :]
    e = jnp.exp(tails)
    r = lax.broadcasted_iota(jnp.int32, (_L, _L), 0)
    c = lax.broadcasted_iota(jnp.int32, (_L, _L), 1)
    tri = jnp.where(c <= r, 1.0, 0.0)
    cs = lax.dot_general(tri, e, (((1,), (0,)), ((), ())),
                         precision=lax.Precision.HIGHEST,
                         preferred_element_type=jnp.float32)
    etot = cs[_L - 1:_L, :]
    others = sum_exp - jnp.exp(tgt) - etot
    below_sum = jnp.sum(jnp.log(others + etot - cs + e), axis=0,
                        keepdims=True)
    above = jnp.sum(tails, axis=0, keepdims=True)
    lpt = above - below_sum
    nl_ref[...] = jnp.log(sum_exp) - tgt - lpt
    lpt_ref[...] = lpt


_tc_tail = pl.pallas_call(
    _tc_tail_body,
    out_shape=[
        jax.ShapeDtypeStruct((1, _B), jnp.float32),
        jax.ShapeDtypeStruct((1, _B), jnp.float32),
    ],
)


_NIDX = _B * (_L + 1)
_NC = 2
_NSUB = 16
_NW = _NC * _NSUB
_PERW = _NIDX // _NW
_NCH = _PERW // 16


def _sc_body(table, fidx_hbm, out_hbm, fidx_v, sel_v, sem):
    wid = lax.axis_index("s") * _NC + lax.axis_index("c")
    base = wid * _PERW
    pltpu.sync_copy(fidx_hbm.at[pl.ds(base, _PERW)], fidx_v)
    pltpu.async_copy(table.at[fidx_v], sel_v, sem).wait()
    pltpu.sync_copy(sel_v, out_hbm.at[pl.ds(base, _PERW)])


@functools.cache
def _sc_gather():
    return functools.partial(
        pl.kernel,
        mesh=plsc.VectorSubcoreMesh(core_axis_name="c", subcore_axis_name="s"),
        out_type=jax.ShapeDtypeStruct((_NIDX,), jnp.float32),
        scratch_types=[
            pltpu.VMEM((_PERW,), jnp.int32),
            pltpu.VMEM((_PERW,), jnp.float32),
            pltpu.SemaphoreType.DMA,
        ],
    )(_sc_body)


_SEGTOT = _NSC // 8
_SEG0 = _R0 // 8


def _sc_sum_body(table, part_hbm, buf_v, acc_v, sem):
    wid = lax.axis_index("s") * _NC + lax.axis_index("c")
    per = _SEGTOT // _NW
    rem = _SEGTOT % _NW
    cnt = per + jnp.where(wid < rem, 1, 0)
    base = _SEG0 + wid * per + jnp.minimum(wid, rem)

    for m in range(64):
        acc_v[pl.ds(m * 16, 16)] = jnp.zeros((16,), jnp.float32)

    def seg_body(s, carry):
        pltpu.sync_copy(table.at[pl.ds((base + s) * 8192, 8192)], buf_v)
        for g in range(64):
            tc, c16 = g >> 3, g & 7
            seg_off = tc * 1024 + c16 * 16
            acc_off = tc * 128 + c16 * 16
            a = acc_v[pl.ds(acc_off, 16)]
            for r in range(8):
                a = a + jnp.exp(buf_v[pl.ds(seg_off + r * 128, 16)])
            acc_v[pl.ds(acc_off, 16)] = a
        return carry

    lax.fori_loop(0, cnt, seg_body, 0)
    pltpu.sync_copy(acc_v, part_hbm.at[wid])


@functools.cache
def _sc_sum():
    return functools.partial(
        pl.kernel,
        mesh=plsc.VectorSubcoreMesh(core_axis_name="c", subcore_axis_name="s"),
        out_type=jax.ShapeDtypeStruct((_NW, _B), jnp.float32),
        scratch_types=[
            pltpu.VMEM((8192,), jnp.float32),
            pltpu.VMEM((_B,), jnp.float32),
            pltpu.SemaphoreType.DMA,
        ],
    )(_sc_sum_body)


def kernel(output, target, tails):
    flat_view = (output.T.reshape(_N // 8, 8, _B // 128, 128)
                 .transpose(0, 2, 1, 3).reshape(-1))
    idx = jnp.concatenate([tails, target[:, None]], axis=1).astype(jnp.int32)
    i_b = jnp.arange(_B, dtype=jnp.int32)[:, None]
    phys = (((idx >> 3) * (_B // 128) + (i_b >> 7)) * 8 + (idx & 7)) * 128 \
        + (i_b & 127)
    g = _sc_gather()(flat_view, phys.T.reshape(-1))
    gt = g.reshape(_L + 1, _B)
    parts = _sc_sum()(flat_view)
    acc8 = _tc_sum(output.T)
    nl, lpt = _tc_tail(acc8, parts, gt)
    return nl[0], lpt[0]
